# baseline (speedup 1.0000x reference)
import functools

import jax
import jax.numpy as jnp
from jax import lax
from jax.experimental import pallas as pl
from jax.experimental.pallas import tpu as pltpu
from jax.experimental.pallas import tpu_sc as plsc

T = 2048
H = 1024
F = 4096
E = 8
K = 2
CAP = 640
NCLS = 10
SLOTS = E * CAP
PAD_SLOTS = SLOTS + 8
LANES = 128

TT = 256
NT = T // TT
FC = 512
NF = F // FC

NW = 32
TPW = T // NW
TPW2 = TPW // 2


def _ka_body(x_ref, Wb_ref, bb_ref, Wg_ref, Wc_ref, bc_ref,
             h_ref, s0_ref, s1_ref, g0s_ref, g1s_ref, w0_ref, w1_ref,
             laux_ref, cnt_ref, c0_ref, c1_ref, base_ref, me_ref, ce_ref):
    ti = pl.program_id(0)

    @pl.when(ti == 0)
    def _():
        base_ref[...] = jnp.zeros((1, LANES), jnp.float32)
        me_ref[...] = jnp.zeros((1, LANES), jnp.float32)
        ce_ref[...] = jnp.zeros((1, LANES), jnp.float32)

    xt = x_ref[...]
    ht = jnp.maximum(
        jnp.dot(xt, Wb_ref[...], preferred_element_type=jnp.float32)
        + bb_ref[...], 0.0)
    h_ref[...] = ht

    lanef = jax.lax.broadcasted_iota(
        jnp.int32, (TT, LANES), 1).astype(jnp.float32)
    elane = lanef < E
    lg = jnp.dot(ht, Wg_ref[...], preferred_element_type=jnp.float32)
    lg = jnp.where(elane, lg, -1e30)
    m = jnp.max(lg, axis=1, keepdims=True)
    el = jnp.where(elane, jnp.exp(lg - m), 0.0)
    probs = el / jnp.sum(el, axis=1, keepdims=True)

    p1 = jnp.max(probs, axis=1, keepdims=True)
    i1 = jnp.min(jnp.where(probs == p1, lanef, 1e9), axis=1, keepdims=True)
    oh0 = (lanef == i1).astype(jnp.float32)
    probs2 = jnp.where(lanef == i1, -1.0, probs)
    p2 = jnp.max(probs2, axis=1, keepdims=True)
    i2 = jnp.min(jnp.where(probs2 == p2, lanef, 1e9), axis=1, keepdims=True)
    oh1 = (lanef == i2).astype(jnp.float32)

    r = jax.lax.broadcasted_iota(jnp.int32, (TT, TT), 0)
    c = jax.lax.broadcasted_iota(jnp.int32, (TT, TT), 1)
    Lst = (r > c).astype(jnp.float32)
    ohsum = oh0 + oh1
    prevc = jnp.dot(Lst, ohsum, preferred_element_type=jnp.float32)
    basec = base_ref[...]
    loc0 = jnp.sum(oh0 * (prevc + basec), axis=1, keepdims=True)
    loc1 = jnp.sum(oh1 * (prevc + basec), axis=1, keepdims=True)

    base_ref[...] = basec + jnp.sum(ohsum, axis=0, keepdims=True)
    me_ref[...] += jnp.sum(probs, axis=0, keepdims=True)
    ce_ref[...] += jnp.sum(oh0, axis=0, keepdims=True)

    p12 = p1 + p2
    w0 = p1 / p12
    w1 = p2 / p12
    v0 = loc0 < CAP
    v1 = loc1 < CAP
    s0 = jnp.where(v0, i1 * CAP + loc0, float(SLOTS))
    s1 = jnp.where(v1, i2 * CAP + loc1, float(SLOTS))
    s0_ref[...] = s0.astype(jnp.int32)
    s1_ref[...] = s1.astype(jnp.int32)
    g0s_ref[...] = jnp.minimum(s0, float(SLOTS - 1)).astype(jnp.int32)
    g1s_ref[...] = jnp.minimum(s1, float(SLOTS - 1)).astype(jnp.int32)
    w0_ref[...] = jnp.where(v0, w0, 0.0)
    w1_ref[...] = jnp.where(v1, w1, 0.0)

    clane = lanef < 2
    cl = jnp.dot(ht, Wc_ref[...], preferred_element_type=jnp.float32)
    cl = jnp.where(clane, cl + bc_ref[...], -1e30)
    cm = jnp.max(cl, axis=1, keepdims=True)
    cexp = jnp.where(clane, jnp.exp(cl - cm), 0.0)
    cprob = cexp / jnp.sum(cexp, axis=1, keepdims=True)
    c0_ref[...] = jnp.sum(jnp.where(lanef == 0, cprob, 0.0),
                          axis=1, keepdims=True)
    c1_ref[...] = jnp.sum(jnp.where(lanef == 1, cprob, 0.0),
                          axis=1, keepdims=True)

    @pl.when(ti == NT - 1)
    def _():
        me = me_ref[...] / T
        ce = ce_ref[...] / T
        laux_ref[...] = E * jnp.sum(me * ce, axis=(0, 1), keepdims=True)
        cnt_ref[...] = base_ref[...].astype(jnp.int32)


def _routing(x, Wb, bb, Wg_pad, Wc_pad, bc_pad):
    col = lambda t: (t, 0)
    fixed = lambda t: (0, 0)
    return pl.pallas_call(
        _ka_body,
        grid=(NT,),
        in_specs=[
            pl.BlockSpec((TT, H), col),
            pl.BlockSpec((H, H), fixed),
            pl.BlockSpec((1, H), fixed),
            pl.BlockSpec((H, LANES), fixed),
            pl.BlockSpec((H, LANES), fixed),
            pl.BlockSpec((1, LANES), fixed),
        ],
        out_specs=[
            pl.BlockSpec((TT, H), col),
            pl.BlockSpec((TT, 1), col),
            pl.BlockSpec((TT, 1), col),
            pl.BlockSpec((TT, 1), col),
            pl.BlockSpec((TT, 1), col),
            pl.BlockSpec((TT, 1), col),
            pl.BlockSpec((TT, 1), col),
            pl.BlockSpec((1, 1), fixed),
            pl.BlockSpec((1, LANES), fixed),
            pl.BlockSpec((TT, 1), col),
            pl.BlockSpec((TT, 1), col),
        ],
        out_shape=[
            jax.ShapeDtypeStruct((T, H), jnp.float32),
            jax.ShapeDtypeStruct((T, 1), jnp.int32),
            jax.ShapeDtypeStruct((T, 1), jnp.int32),
            jax.ShapeDtypeStruct((T, 1), jnp.int32),
            jax.ShapeDtypeStruct((T, 1), jnp.int32),
            jax.ShapeDtypeStruct((T, 1), jnp.float32),
            jax.ShapeDtypeStruct((T, 1), jnp.float32),
            jax.ShapeDtypeStruct((1, 1), jnp.float32),
            jax.ShapeDtypeStruct((1, LANES), jnp.int32),
            jax.ShapeDtypeStruct((T, 1), jnp.float32),
            jax.ShapeDtypeStruct((T, 1), jnp.float32),
        ],
        scratch_shapes=[
            pltpu.VMEM((1, LANES), jnp.float32),
            pltpu.VMEM((1, LANES), jnp.float32),
            pltpu.VMEM((1, LANES), jnp.float32),
        ],
    )(x, Wb, bb, Wg_pad, Wc_pad, bc_pad)


def _sc_dispatch(h, s0w, s1w):
    mesh = plsc.VectorSubcoreMesh(core_axis_name="c", subcore_axis_name="s")

    @functools.partial(
        pl.kernel, mesh=mesh,
        out_type=jax.ShapeDtypeStruct((PAD_SLOTS, H), jnp.float32),
        scratch_types=[
            pltpu.VMEM((TPW,), jnp.int32),
            pltpu.VMEM((TPW,), jnp.int32),
            pltpu.VMEM((TPW, H), jnp.float32),
            pltpu.SemaphoreType.DMA,
        ],
    )
    def run(h_hbm, s0_hbm, s1_hbm, buf_hbm, idx0_v, idx1_v, rows_v, sem):
        wid = lax.axis_index("s") * 2 + lax.axis_index("c")
        base = wid * TPW
        pltpu.sync_copy(s0_hbm.at[wid], idx0_v)
        pltpu.sync_copy(s1_hbm.at[wid], idx1_v)
        pltpu.sync_copy(h_hbm.at[pl.ds(base, TPW)], rows_v)
        pltpu.async_copy(rows_v, buf_hbm.at[idx0_v], sem).wait()
        pltpu.async_copy(rows_v, buf_hbm.at[idx1_v], sem).wait()

    return run(h, s0w, s1w)


def _kc_body(cnt_ref, buf_ref, W1_ref, b1_ref, W2_ref, b2_ref, eout_ref,
             bufb_ref, acc_ref):
    e = pl.program_id(0)
    nf = pl.program_id(1)

    @pl.when(nf == 0)
    def _():
        load = jnp.minimum(cnt_ref[e], CAP)
        rowi = jax.lax.broadcasted_iota(jnp.int32, (CAP, 1), 0)
        bufb_ref[...] = jnp.where(rowi < load, buf_ref[...], 0.0)

    hmid = jnp.maximum(
        jnp.dot(bufb_ref[...], W1_ref[0], preferred_element_type=jnp.float32)
        + b1_ref[0], 0.0)
    contrib = jnp.dot(hmid, W2_ref[0], preferred_element_type=jnp.float32)

    @pl.when(nf == 0)
    def _():
        acc_ref[...] = contrib + b2_ref[0]

    @pl.when(nf > 0)
    def _():
        acc_ref[...] += contrib

    @pl.when(nf == NF - 1)
    def _():
        eout_ref[...] = acc_ref[...]


def _expert_ffn(cnt, buf, W1, b1r, W2, b2r):
    return pl.pallas_call(
        _kc_body,
        grid=(E, NF),
        in_specs=[
            pl.BlockSpec(memory_space=pltpu.SMEM),
            pl.BlockSpec((CAP, H), lambda e, nf: (e, 0)),
            pl.BlockSpec((1, H, FC), lambda e, nf: (e, 0, nf)),
            pl.BlockSpec((1, 1, FC), lambda e, nf: (e, 0, nf)),
            pl.BlockSpec((1, FC, H), lambda e, nf: (e, nf, 0)),
            pl.BlockSpec((1, 1, H), lambda e, nf: (e, 0, 0)),
        ],
        out_specs=pl.BlockSpec((CAP, H), lambda e, nf: (e, 0)),
        out_shape=jax.ShapeDtypeStruct((SLOTS, H), jnp.float32),
        scratch_shapes=[pltpu.VMEM((CAP, H), jnp.float32),
                        pltpu.VMEM((CAP, H), jnp.float32)],
        compiler_params=pltpu.CompilerParams(
            dimension_semantics=("arbitrary", "arbitrary"),
            vmem_limit_bytes=100 * 1024 * 1024),
    )(cnt, buf, W1, b1r, W2, b2r)


def _sc_combine(eout, g0w, g1w):
    mesh = plsc.VectorSubcoreMesh(core_axis_name="c", subcore_axis_name="s")

    @functools.partial(
        pl.kernel, mesh=mesh,
        out_type=[
            jax.ShapeDtypeStruct((T, H), jnp.float32),
            jax.ShapeDtypeStruct((T, H), jnp.float32),
        ],
        scratch_types=[
            pltpu.VMEM((TPW,), jnp.int32),
            pltpu.VMEM((TPW, H), jnp.float32),
            pltpu.SemaphoreType.DMA,
        ],
    )
    def run(eout_hbm, g0_hbm, g1_hbm, o0_hbm, o1_hbm, idx_v, rows_v, sem):
        wid = lax.axis_index("s") * 2 + lax.axis_index("c")
        base = wid * TPW
        pltpu.sync_copy(g0_hbm.at[wid], idx_v)
        pltpu.async_copy(eout_hbm.at[idx_v], rows_v, sem).wait()
        pltpu.sync_copy(rows_v, o0_hbm.at[pl.ds(base, TPW)])
        pltpu.sync_copy(g1_hbm.at[wid], idx_v)
        pltpu.async_copy(eout_hbm.at[idx_v], rows_v, sem).wait()
        pltpu.sync_copy(rows_v, o1_hbm.at[pl.ds(base, TPW)])

    return run(eout, g0w, g1w)


SUBT = 512
NSUB = T // SUBT


def _ke_body(h_ref, Wr1_ref, br1_ref, Wr2_ref, br2_ref, res_ref, acc_ref):
    nf = pl.program_id(0)
    for tt in range(NSUB):
        hs = h_ref[pl.ds(tt * SUBT, SUBT), :]
        mid = jnp.maximum(
            jnp.dot(hs, Wr1_ref[...], preferred_element_type=jnp.float32)
            + br1_ref[...], 0.0)
        contrib = jnp.dot(mid, Wr2_ref[...],
                          preferred_element_type=jnp.float32)

        @pl.when(nf == 0)
        def _():
            acc_ref[pl.ds(tt * SUBT, SUBT), :] = contrib + br2_ref[...]

        @pl.when(nf > 0)
        def _():
            acc_ref[pl.ds(tt * SUBT, SUBT), :] += contrib

    @pl.when(nf == NF - 1)
    def _():
        res_ref[...] = acc_ref[...]


def _residual(h, Wr1, br1r, Wr2, br2r):
    fixed = lambda nf: (0, 0)
    return pl.pallas_call(
        _ke_body,
        grid=(NF,),
        in_specs=[
            pl.BlockSpec((T, H), fixed),
            pl.BlockSpec((H, FC), lambda nf: (0, nf)),
            pl.BlockSpec((1, FC), lambda nf: (0, nf)),
            pl.BlockSpec((FC, H), lambda nf: (nf, 0)),
            pl.BlockSpec((1, H), fixed),
        ],
        out_specs=pl.BlockSpec((T, H), fixed),
        out_shape=jax.ShapeDtypeStruct((T, H), jnp.float32),
        scratch_shapes=[pltpu.VMEM((T, H), jnp.float32)],
        compiler_params=pltpu.CompilerParams(
            dimension_semantics=("arbitrary",),
            vmem_limit_bytes=62 * 1024 * 1024),
    )(h, Wr1, br1r, Wr2, br2r)


def _kf_body(res_ref, ge0_ref, ge1_ref, w0_ref, w1_ref, c0_ref, c1_ref,
             Wh_ref, bh_ref, out_ref):
    moe = ge0_ref[...] * w0_ref[...] + ge1_ref[...] * w1_ref[...]
    comb = moe * c0_ref[...] + res_ref[...] * c1_ref[...]
    out_ref[...] = (
        jnp.dot(comb, Wh_ref[...], preferred_element_type=jnp.float32)
        + bh_ref[...])


def _final(res, ge0, ge1, w0, w1, c0, c1, Wh_pad, bh_pad):
    col = lambda t: (t, 0)
    fixed = lambda t: (0, 0)
    return pl.pallas_call(
        _kf_body,
        grid=(NT,),
        in_specs=[
            pl.BlockSpec((TT, H), col),
            pl.BlockSpec((TT, H), col),
            pl.BlockSpec((TT, H), col),
            pl.BlockSpec((TT, 1), col),
            pl.BlockSpec((TT, 1), col),
            pl.BlockSpec((TT, 1), col),
            pl.BlockSpec((TT, 1), col),
            pl.BlockSpec((H, LANES), fixed),
            pl.BlockSpec((1, LANES), fixed),
        ],
        out_specs=pl.BlockSpec((TT, LANES), col),
        out_shape=jax.ShapeDtypeStruct((T, LANES), jnp.float32),
        compiler_params=pltpu.CompilerParams(
            vmem_limit_bytes=62 * 1024 * 1024),
    )(res, ge0, ge1, w0, w1, c0, c1, Wh_pad, bh_pad)


def kernel(x, Wb, bb, Wg, W1, b1, W2, b2, Wr1, br1, Wr2, br2, Wc, bc, Wh, bh):
    bbr = bb.reshape(1, H)
    Wg_pad = jnp.pad(Wg, ((0, 0), (0, LANES - E)))
    Wc_pad = jnp.pad(Wc, ((0, 0), (0, LANES - 2)))
    bc_pad = jnp.pad(bc, (0, LANES - 2)).reshape(1, LANES)
    (h, s0, s1, g0s, g1s, w0, w1, laux, cnt, c0, c1) = _routing(
        x, Wb, bbr, Wg_pad, Wc_pad, bc_pad)

    buf = _sc_dispatch(h, s0.reshape(NW, TPW), s1.reshape(NW, TPW))

    res = _residual(h, Wr1, br1.reshape(1, F), Wr2, br2.reshape(1, H))

    eout = _expert_ffn(cnt.reshape(LANES)[:E], buf[:SLOTS], W1,
                       b1.reshape(E, 1, F), W2, b2.reshape(E, 1, H))

    ge0, ge1 = _sc_combine(eout, g0s.reshape(NW, TPW), g1s.reshape(NW, TPW))

    Wh_pad = jnp.pad(Wh, ((0, 0), (0, LANES - NCLS)))
    bh_pad = jnp.pad(bh, (0, LANES - NCLS)).reshape(1, LANES)
    out_pad = _final(res, ge0, ge1, w0, w1, c0, c1, Wh_pad, bh_pad)

    return out_pad[:, :NCLS], laux.reshape(())
